# baseline (speedup 1.0000x reference)
import jax
import jax.numpy as jnp
from jax.experimental import pallas as pl
from jax.experimental.pallas import tpu as pltpu

B, S, D = 4, 576, 64
K = 8192
TOPK = 8
TEMP_MIN = 0.1
N = B * S
TILE = 128
GRID = N // TILE


def _fused_body(x_ref, gum_ref, cb_ref, itau_ref, scale_ref,
                soft_ref, hard_ref, out_ref, ent_ref, cbn_ref):
    i = pl.program_id(0)

    @pl.when(i == 0)
    def _init():
        cb = cb_ref[...]
        n = jnp.maximum(jnp.sqrt(jnp.sum(cb * cb, axis=-1, keepdims=True)), 1e-12)
        cbn_ref[...] = cb / n
        ent_ref[...] = jnp.zeros((1, 1), jnp.float32)

    inv_tau = itau_ref[0, 0]
    scale = scale_ref[0, 0]

    x = x_ref[...]
    xden = jnp.maximum(jnp.sqrt(jnp.sum(x * x, axis=-1, keepdims=True)), 1e-12)
    xn = x / xden
    logits = jax.lax.dot_general(
        xn, cbn_ref[...], (((1,), (1,)), ((), ())),
        preferred_element_type=jnp.float32)
    y = (logits + gum_ref[...]) * inv_tau

    ones_k = jnp.ones((K, 1), jnp.float32)

    def _rowsum_mxu(a):
        return jax.lax.dot_general(
            a, ones_k, (((1,), (0,)), ((), ())),
            preferred_element_type=jnp.float32,
            precision=jax.lax.Precision.HIGHEST)

    m = jnp.max(y, axis=-1, keepdims=True)
    e = jnp.exp(y - m)
    ssum = _rowsum_mxu(e)
    soft = e * (1.0 / ssum)
    soft_ref[...] = soft
    ent_rows = _rowsum_mxu(soft * jnp.log(soft + 1e-8))
    ent_ref[...] += -jnp.sum(ent_rows, axis=0, keepdims=True)

    neg = jnp.float32(-jnp.inf)
    t8 = m
    yw = y
    for _ in range(TOPK - 1):
        yw = jnp.where(yw < t8, yw, neg)
        t8 = jnp.max(yw, axis=-1, keepdims=True)
    hard = jnp.where(y >= t8, 1.0, 0.0)
    cnt = _rowsum_mxu(hard)
    hard_ref[...] = hard

    @pl.when(jnp.max(cnt) > 8.0)
    def _repair():
        col = jax.lax.broadcasted_iota(jnp.int32, (TILE, K), 1)
        yy = y
        hd = jnp.zeros((TILE, K), jnp.float32)
        for _ in range(TOPK):
            mx = jnp.max(yy, axis=-1, keepdims=True)
            cand = jnp.where(yy == mx, col, K)
            amin = jnp.min(cand, axis=-1, keepdims=True)
            pick = col == amin
            hd = jnp.where(pick, 1.0, hd)
            yy = jnp.where(pick, neg, yy)
        hard_ref[...] = hd

    out_ref[...] = jax.lax.dot_general(
        hard_ref[...], cb_ref[...], (((1,), (0,)), ((), ())),
        preferred_element_type=jnp.float32) * scale


def kernel(x, codebook, output_scale, temperature_raw, gumbel):
    tau = jnp.clip(temperature_raw, TEMP_MIN, None)
    inv_tau = (1.0 / tau).reshape(1, 1).astype(jnp.float32)
    scale = output_scale.reshape(1, 1).astype(jnp.float32)
    x2 = x.reshape(N, D)
    g2 = gumbel.reshape(N, K)

    soft, hard, out, ent = pl.pallas_call(
        _fused_body,
        grid=(GRID,),
        in_specs=[
            pl.BlockSpec((TILE, D), lambda i: (i, 0)),
            pl.BlockSpec((TILE, K), lambda i: (i, 0)),
            pl.BlockSpec((K, D), lambda i: (0, 0)),
            pl.BlockSpec((1, 1), lambda i: (0, 0), memory_space=pltpu.SMEM),
            pl.BlockSpec((1, 1), lambda i: (0, 0), memory_space=pltpu.SMEM),
        ],
        out_specs=[
            pl.BlockSpec((TILE, K), lambda i: (i, 0)),
            pl.BlockSpec((TILE, K), lambda i: (i, 0)),
            pl.BlockSpec((TILE, D), lambda i: (i, 0)),
            pl.BlockSpec((1, 1), lambda i: (0, 0)),
        ],
        out_shape=[
            jax.ShapeDtypeStruct((N, K), jnp.float32),
            jax.ShapeDtypeStruct((N, K), jnp.float32),
            jax.ShapeDtypeStruct((N, D), jnp.float32),
            jax.ShapeDtypeStruct((1, 1), jnp.float32),
        ],
        scratch_shapes=[pltpu.VMEM((K, D), jnp.float32)],
    )(x2, g2, codebook, inv_tau, scale)

    output = out.reshape(B, S, D)
    entropy = ent[0, 0] / N
    return (output, soft.reshape(B, S, K), hard.reshape(B, S, K), entropy)


# speedup vs baseline: 1.9573x; 1.9573x over previous
import jax
import jax.numpy as jnp
from jax.experimental import pallas as pl
from jax.experimental.pallas import tpu as pltpu

B, S, D = 4, 576, 64
K = 8192
TOPK = 8
TEMP_MIN = 0.1
N = B * S
TILE = 128
GRID = N // TILE


def _fused_body(x_ref, gum_ref, cb_ref, itau_ref, scale_ref,
                soft_ref, hard_ref, out_ref, ent_ref, cbn_ref):
    i = pl.program_id(0)

    @pl.when(i == 0)
    def _init():
        cb = cb_ref[...]
        n = jnp.maximum(jnp.sqrt(jnp.sum(cb * cb, axis=-1, keepdims=True)), 1e-12)
        cbn_ref[...] = cb / n
        ent_ref[...] = jnp.zeros((1, 1), jnp.float32)

    inv_tau = itau_ref[0, 0]
    scale = scale_ref[0, 0]

    x = x_ref[...]
    xden = jnp.maximum(jnp.sqrt(jnp.sum(x * x, axis=-1, keepdims=True)), 1e-12)
    xn = x / xden
    logits = jax.lax.dot_general(
        xn, cbn_ref[...], (((1,), (1,)), ((), ())),
        preferred_element_type=jnp.float32)
    y = (logits + gum_ref[...]) * inv_tau

    m = jnp.max(y, axis=-1, keepdims=True)
    e = jnp.exp(y - m)
    ssum = jnp.sum(e, axis=-1, keepdims=True)
    soft = e * (1.0 / ssum)
    soft_ref[...] = soft
    ent_tile = jnp.sum(jnp.sum(soft * jnp.log(soft + 1e-8), axis=1, keepdims=True),
                       axis=0, keepdims=True)
    ent_ref[...] += -ent_tile

    neg = jnp.float32(-jnp.inf)
    t8 = m
    yw = y
    for _ in range(TOPK - 1):
        yw = jnp.where(yw < t8, yw, neg)
        t8 = jnp.max(yw, axis=-1, keepdims=True)
    hard = jnp.where(y >= t8, 1.0, 0.0)
    cnt = jnp.sum(hard, axis=-1, keepdims=True)
    hard_ref[...] = hard

    @pl.when(jnp.max(cnt) > 8.0)
    def _repair():
        col = jax.lax.broadcasted_iota(jnp.int32, (TILE, K), 1)
        yy = y
        hd = jnp.zeros((TILE, K), jnp.float32)
        for _ in range(TOPK):
            mx = jnp.max(yy, axis=-1, keepdims=True)
            cand = jnp.where(yy == mx, col, K)
            amin = jnp.min(cand, axis=-1, keepdims=True)
            pick = col == amin
            hd = jnp.where(pick, 1.0, hd)
            yy = jnp.where(pick, neg, yy)
        hard_ref[...] = hd

    out_ref[...] = jax.lax.dot_general(
        hard_ref[...], cb_ref[...], (((1,), (0,)), ((), ())),
        preferred_element_type=jnp.float32) * scale


def kernel(x, codebook, output_scale, temperature_raw, gumbel):
    tau = jnp.clip(temperature_raw, TEMP_MIN, None)
    inv_tau = (1.0 / tau).reshape(1, 1).astype(jnp.float32)
    scale = output_scale.reshape(1, 1).astype(jnp.float32)
    x2 = x.reshape(N, D)
    g2 = gumbel.reshape(N, K)

    soft, hard, out, ent = pl.pallas_call(
        _fused_body,
        grid=(GRID,),
        in_specs=[
            pl.BlockSpec((TILE, D), lambda i: (i, 0)),
            pl.BlockSpec((TILE, K), lambda i: (i, 0)),
            pl.BlockSpec((K, D), lambda i: (0, 0)),
            pl.BlockSpec((1, 1), lambda i: (0, 0), memory_space=pltpu.SMEM),
            pl.BlockSpec((1, 1), lambda i: (0, 0), memory_space=pltpu.SMEM),
        ],
        out_specs=[
            pl.BlockSpec((TILE, K), lambda i: (i, 0)),
            pl.BlockSpec((TILE, K), lambda i: (i, 0)),
            pl.BlockSpec((TILE, D), lambda i: (i, 0)),
            pl.BlockSpec((1, 1), lambda i: (0, 0)),
        ],
        out_shape=[
            jax.ShapeDtypeStruct((N, K), jnp.float32),
            jax.ShapeDtypeStruct((N, K), jnp.float32),
            jax.ShapeDtypeStruct((N, D), jnp.float32),
            jax.ShapeDtypeStruct((1, 1), jnp.float32),
        ],
        scratch_shapes=[pltpu.VMEM((K, D), jnp.float32)],
    )(x2, g2, codebook, inv_tau, scale)

    output = out.reshape(B, S, D)
    entropy = ent[0, 0] / N
    return (output, soft.reshape(B, S, K), hard.reshape(B, S, K), entropy)


# speedup vs baseline: 2.5649x; 1.3104x over previous
import jax
import jax.numpy as jnp
from jax.experimental import pallas as pl
from jax.experimental.pallas import tpu as pltpu

B, S, D = 4, 576, 64
K = 8192
TOPK = 8
TEMP_MIN = 0.1
N = B * S
TILE = 128
GRID = N // TILE


def _fused_body(x_ref, gum_ref, cb_ref, itau_ref, scale_ref,
                soft_ref, hard_ref, out_ref, ent_ref, cbn_ref):
    i = pl.program_id(0)

    @pl.when(i == 0)
    def _init():
        cb = cb_ref[...]
        n = jnp.maximum(jnp.sqrt(jnp.sum(cb * cb, axis=-1, keepdims=True)), 1e-12)
        cbn_ref[...] = cb / n
        ent_ref[...] = jnp.zeros((1, 1), jnp.float32)

    inv_tau = itau_ref[0, 0]
    scale = scale_ref[0, 0]

    x = x_ref[...]
    xden = jnp.maximum(jnp.sqrt(jnp.sum(x * x, axis=-1, keepdims=True)), 1e-12)
    xn = x / xden
    logits = jax.lax.dot_general(
        xn, cbn_ref[...], (((1,), (1,)), ((), ())),
        preferred_element_type=jnp.float32)
    y = (logits + gum_ref[...]) * inv_tau

    sl = [y[:, j * 128:(j + 1) * 128] for j in range(K // 128)]
    lists2 = []
    for j in range(len(sl) // 2):
        a, b = sl[2 * j], sl[2 * j + 1]
        lists2.append((jnp.maximum(a, b), jnp.minimum(a, b)))

    def _merge22(A, B):
        a0, a1 = A
        b0, b1 = B
        h0 = jnp.maximum(a0, b1)
        l0 = jnp.minimum(a0, b1)
        h1 = jnp.maximum(a1, b0)
        l1 = jnp.minimum(a1, b0)
        return (jnp.maximum(h0, h1), jnp.minimum(h0, h1),
                jnp.maximum(l0, l1), jnp.minimum(l0, l1))

    def _merge44(A, B, sort):
        m0 = jnp.maximum(A[0], B[3])
        m1 = jnp.maximum(A[1], B[2])
        m2 = jnp.maximum(A[2], B[1])
        m3 = jnp.maximum(A[3], B[0])
        if not sort:
            return (m0, m1, m2, m3)
        h0 = jnp.maximum(m0, m2)
        l0 = jnp.minimum(m0, m2)
        h1 = jnp.maximum(m1, m3)
        l1 = jnp.minimum(m1, m3)
        return (jnp.maximum(h0, h1), jnp.minimum(h0, h1),
                jnp.maximum(l0, l1), jnp.minimum(l0, l1))

    cur = [_merge22(lists2[2 * j], lists2[2 * j + 1])
           for j in range(len(lists2) // 2)]
    while len(cur) > 1:
        cur = [_merge44(cur[2 * j], cur[2 * j + 1], sort=len(cur) > 2)
               for j in range(len(cur) // 2)]
    cand = jnp.concatenate(cur[0], axis=1)

    m = jnp.max(cand, axis=-1, keepdims=True)
    e = jnp.exp(y - m)
    ssum = jnp.sum(e, axis=-1, keepdims=True)
    soft = e * (1.0 / ssum)
    soft_ref[...] = soft
    ent_tile = jnp.sum(jnp.sum(soft * jnp.log(soft + 1e-8), axis=1, keepdims=True),
                       axis=0, keepdims=True)
    ent_ref[...] += -ent_tile

    neg = jnp.float32(-jnp.inf)
    t8 = m
    cw = cand
    for _ in range(TOPK - 1):
        cw = jnp.where(cw < t8, cw, neg)
        t8 = jnp.max(cw, axis=-1, keepdims=True)
    hard = jnp.where(y >= t8, 1.0, 0.0)
    cnt = jnp.sum(hard, axis=-1, keepdims=True)
    hard_ref[...] = hard

    @pl.when(jnp.max(cnt) > 8.0)
    def _repair():
        col = jax.lax.broadcasted_iota(jnp.int32, (TILE, K), 1)
        yy = y
        hd = jnp.zeros((TILE, K), jnp.float32)
        for _ in range(TOPK):
            mx = jnp.max(yy, axis=-1, keepdims=True)
            cand = jnp.where(yy == mx, col, K)
            amin = jnp.min(cand, axis=-1, keepdims=True)
            pick = col == amin
            hd = jnp.where(pick, 1.0, hd)
            yy = jnp.where(pick, neg, yy)
        hard_ref[...] = hd

    out_ref[...] = jax.lax.dot_general(
        hard_ref[...], cb_ref[...], (((1,), (0,)), ((), ())),
        preferred_element_type=jnp.float32) * scale


def kernel(x, codebook, output_scale, temperature_raw, gumbel):
    tau = jnp.clip(temperature_raw, TEMP_MIN, None)
    inv_tau = (1.0 / tau).reshape(1, 1).astype(jnp.float32)
    scale = output_scale.reshape(1, 1).astype(jnp.float32)
    x2 = x.reshape(N, D)
    g2 = gumbel.reshape(N, K)

    soft, hard, out, ent = pl.pallas_call(
        _fused_body,
        grid=(GRID,),
        in_specs=[
            pl.BlockSpec((TILE, D), lambda i: (i, 0)),
            pl.BlockSpec((TILE, K), lambda i: (i, 0)),
            pl.BlockSpec((K, D), lambda i: (0, 0)),
            pl.BlockSpec((1, 1), lambda i: (0, 0), memory_space=pltpu.SMEM),
            pl.BlockSpec((1, 1), lambda i: (0, 0), memory_space=pltpu.SMEM),
        ],
        out_specs=[
            pl.BlockSpec((TILE, K), lambda i: (i, 0)),
            pl.BlockSpec((TILE, K), lambda i: (i, 0)),
            pl.BlockSpec((TILE, D), lambda i: (i, 0)),
            pl.BlockSpec((1, 1), lambda i: (0, 0)),
        ],
        out_shape=[
            jax.ShapeDtypeStruct((N, K), jnp.float32),
            jax.ShapeDtypeStruct((N, K), jnp.float32),
            jax.ShapeDtypeStruct((N, D), jnp.float32),
            jax.ShapeDtypeStruct((1, 1), jnp.float32),
        ],
        scratch_shapes=[pltpu.VMEM((K, D), jnp.float32)],
    )(x2, g2, codebook, inv_tau, scale)

    output = out.reshape(B, S, D)
    entropy = ent[0, 0] / N
    return (output, soft.reshape(B, S, K), hard.reshape(B, S, K), entropy)


# speedup vs baseline: 2.6783x; 1.0442x over previous
import jax
import jax.numpy as jnp
from jax.experimental import pallas as pl
from jax.experimental.pallas import tpu as pltpu

B, S, D = 4, 576, 64
K = 8192
TOPK = 8
TEMP_MIN = 0.1
N = B * S
TILE = 128
GRID = N // TILE


def _fused_body(x_ref, gum_ref, cb_ref, itau_ref, scale_ref,
                soft_ref, hard_ref, out_ref, ent_ref, cbn_ref):
    i = pl.program_id(0)

    @pl.when(i == 0)
    def _init():
        cb = cb_ref[...]
        n = jnp.maximum(jnp.sqrt(jnp.sum(cb * cb, axis=-1, keepdims=True)), 1e-12)
        cbn_ref[...] = cb / n
        ent_ref[...] = jnp.zeros((1, 1), jnp.float32)

    inv_tau = itau_ref[0, 0]
    scale = scale_ref[0, 0]

    x = x_ref[...]
    xden = jnp.maximum(jnp.sqrt(jnp.sum(x * x, axis=-1, keepdims=True)), 1e-12)
    xn = x / xden
    logits = jax.lax.dot_general(
        xn, cbn_ref[...], (((1,), (1,)), ((), ())),
        preferred_element_type=jnp.float32)
    y = (logits + gum_ref[...]) * inv_tau

    sl = [y[:, j * 128:(j + 1) * 128] for j in range(K // 128)]
    lists2 = []
    for j in range(len(sl) // 2):
        a, b = sl[2 * j], sl[2 * j + 1]
        lists2.append((jnp.maximum(a, b), jnp.minimum(a, b)))

    def _merge22(A, B):
        a0, a1 = A
        b0, b1 = B
        h0 = jnp.maximum(a0, b1)
        l0 = jnp.minimum(a0, b1)
        h1 = jnp.maximum(a1, b0)
        l1 = jnp.minimum(a1, b0)
        return (jnp.maximum(h0, h1), jnp.minimum(h0, h1),
                jnp.maximum(l0, l1), jnp.minimum(l0, l1))

    def _merge44(A, B, sort):
        m0 = jnp.maximum(A[0], B[3])
        m1 = jnp.maximum(A[1], B[2])
        m2 = jnp.maximum(A[2], B[1])
        m3 = jnp.maximum(A[3], B[0])
        if not sort:
            return (m0, m1, m2, m3)
        h0 = jnp.maximum(m0, m2)
        l0 = jnp.minimum(m0, m2)
        h1 = jnp.maximum(m1, m3)
        l1 = jnp.minimum(m1, m3)
        return (jnp.maximum(h0, h1), jnp.minimum(h0, h1),
                jnp.maximum(l0, l1), jnp.minimum(l0, l1))

    cur = [_merge22(lists2[2 * j], lists2[2 * j + 1])
           for j in range(len(lists2) // 2)]
    while len(cur) > 1:
        cur = [_merge44(cur[2 * j], cur[2 * j + 1], sort=len(cur) > 2)
               for j in range(len(cur) // 2)]
    cand = jnp.concatenate(cur[0], axis=1)

    m = jnp.max(cand, axis=-1, keepdims=True)
    w = y - m
    e = jnp.exp(w)
    ssum = jnp.sum(e, axis=-1, keepdims=True)
    soft = e * (1.0 / ssum)
    soft_ref[...] = soft
    ew_sum = jnp.sum(e * w, axis=-1, keepdims=True)
    ent_rows = ew_sum * (1.0 / ssum) - jnp.log(ssum)
    ent_ref[...] += -jnp.sum(ent_rows, axis=0, keepdims=True)

    neg = jnp.float32(-jnp.inf)
    t8 = m
    cw = cand
    for _ in range(TOPK - 1):
        cw = jnp.where(cw < t8, cw, neg)
        t8 = jnp.max(cw, axis=-1, keepdims=True)
    hard = jnp.where(y >= t8, 1.0, 0.0)
    cnt = jnp.sum(hard, axis=-1, keepdims=True)
    hard_ref[...] = hard

    @pl.when(jnp.max(cnt) > 8.0)
    def _repair():
        col = jax.lax.broadcasted_iota(jnp.int32, (TILE, K), 1)
        yy = y
        hd = jnp.zeros((TILE, K), jnp.float32)
        for _ in range(TOPK):
            mx = jnp.max(yy, axis=-1, keepdims=True)
            cand = jnp.where(yy == mx, col, K)
            amin = jnp.min(cand, axis=-1, keepdims=True)
            pick = col == amin
            hd = jnp.where(pick, 1.0, hd)
            yy = jnp.where(pick, neg, yy)
        hard_ref[...] = hd

    out_ref[...] = jax.lax.dot_general(
        hard_ref[...], cb_ref[...], (((1,), (0,)), ((), ())),
        preferred_element_type=jnp.float32) * scale


def kernel(x, codebook, output_scale, temperature_raw, gumbel):
    tau = jnp.clip(temperature_raw, TEMP_MIN, None)
    inv_tau = (1.0 / tau).reshape(1, 1).astype(jnp.float32)
    scale = output_scale.reshape(1, 1).astype(jnp.float32)
    x2 = x.reshape(N, D)
    g2 = gumbel.reshape(N, K)

    soft, hard, out, ent = pl.pallas_call(
        _fused_body,
        grid=(GRID,),
        in_specs=[
            pl.BlockSpec((TILE, D), lambda i: (i, 0)),
            pl.BlockSpec((TILE, K), lambda i: (i, 0)),
            pl.BlockSpec((K, D), lambda i: (0, 0)),
            pl.BlockSpec((1, 1), lambda i: (0, 0), memory_space=pltpu.SMEM),
            pl.BlockSpec((1, 1), lambda i: (0, 0), memory_space=pltpu.SMEM),
        ],
        out_specs=[
            pl.BlockSpec((TILE, K), lambda i: (i, 0)),
            pl.BlockSpec((TILE, K), lambda i: (i, 0)),
            pl.BlockSpec((TILE, D), lambda i: (i, 0)),
            pl.BlockSpec((1, 1), lambda i: (0, 0)),
        ],
        out_shape=[
            jax.ShapeDtypeStruct((N, K), jnp.float32),
            jax.ShapeDtypeStruct((N, K), jnp.float32),
            jax.ShapeDtypeStruct((N, D), jnp.float32),
            jax.ShapeDtypeStruct((1, 1), jnp.float32),
        ],
        scratch_shapes=[pltpu.VMEM((K, D), jnp.float32)],
    )(x2, g2, codebook, inv_tau, scale)

    output = out.reshape(B, S, D)
    entropy = ent[0, 0] / N
    return (output, soft.reshape(B, S, K), hard.reshape(B, S, K), entropy)
